# baseline (speedup 1.0000x reference)
import functools
import jax
import jax.numpy as jnp
from jax import lax
from jax.experimental import pallas as pl
from jax.experimental.pallas import tpu as pltpu
from jax.experimental.pallas import tpu_sc as plsc

N = 10000
E = 320000
D = 128
NPAD = 10240
ROWS_PER_TILE = NPAD // 16
NBLK0 = 80
NBLK1 = 80
EPAD = 16 * (NBLK0 + NBLK1) * 128
EROWS = EPAD // 128
DEG_EPW = EPAD // 32
PADIDX = NPAD - 1

_mesh = plsc.VectorSubcoreMesh(core_axis_name="c", subcore_axis_name="s")


@functools.partial(
    pl.kernel,
    mesh=_mesh,
    out_type=jax.ShapeDtypeStruct((2, NPAD), jnp.float32),
    scratch_types=[
        pltpu.VMEM((DEG_EPW,), jnp.int32),
        pltpu.VMEM((NPAD,), jnp.float32),
        pltpu.VMEM((ROWS_PER_TILE,), jnp.float32),
        pltpu.VMEM((ROWS_PER_TILE,), jnp.float32),
        pltpu.VMEM_SHARED((16, NPAD), jnp.float32),
    ],
    compiler_params=pltpu.CompilerParams(needs_layout_passes=False),
)
def _deg_kernel(pk_hbm, deg_out, pk_v, deg_v, tmp_v, red_v, slots):
    c = lax.axis_index("c")
    s = lax.axis_index("s")
    wid = c * 16 + s
    pltpu.sync_copy(pk_hbm.at[wid], pk_v)

    zeros16 = jnp.zeros((16,), jnp.float32)

    @pl.loop(0, NPAD // 16)
    def _(i):
        deg_v[pl.ds(i * 16, 16)] = zeros16

    @pl.loop(0, DEG_EPW // 16)
    def _(i):
        idx = lax.shift_right_logical(pk_v[pl.ds(i * 16, 16)], 14)
        cnt, last = plsc.scan_count(idx)
        plsc.addupdate_scatter(
            deg_v, [idx], cnt.astype(jnp.float32), mask=last)

    pltpu.sync_copy(deg_v, slots.at[s])
    plsc.subcore_barrier()

    @pl.loop(0, ROWS_PER_TILE // 16)
    def _(i):
        red_v[pl.ds(i * 16, 16)] = zeros16

    @pl.loop(0, 16)
    def _(t):
        pltpu.sync_copy(slots.at[t, pl.ds(s * ROWS_PER_TILE, ROWS_PER_TILE)],
                        tmp_v)

        @pl.loop(0, ROWS_PER_TILE // 16)
        def _(i):
            sl = pl.ds(i * 16, 16)
            red_v[sl] = red_v[sl] + tmp_v[sl]

    pltpu.sync_copy(red_v,
                    deg_out.at[c, pl.ds(s * ROWS_PER_TILE, ROWS_PER_TILE)])


def _scale_body(x_ref, w_ref, d0_ref, d1_ref, scaled_ref, dis_ref):
    deg = d0_ref[...] + d1_ref[...] + 1.0
    dis = lax.rsqrt(deg)
    xw = jnp.dot(x_ref[...], w_ref[...], preferred_element_type=jnp.float32)
    scaled_ref[...] = xw * dis
    dis_ref[...] = dis


def _scale_call(x_pad, W, deg0, deg1):
    blk = 512
    grid = NPAD // blk
    return pl.pallas_call(
        _scale_body,
        grid=(grid,),
        in_specs=[
            pl.BlockSpec((blk, D), lambda i: (i, 0)),
            pl.BlockSpec((D, D), lambda i: (0, 0)),
            pl.BlockSpec((blk, 1), lambda i: (i, 0)),
            pl.BlockSpec((blk, 1), lambda i: (i, 0)),
        ],
        out_specs=[
            pl.BlockSpec((blk, D), lambda i: (i, 0)),
            pl.BlockSpec((blk, 1), lambda i: (i, 0)),
        ],
        out_shape=[
            jax.ShapeDtypeStruct((NPAD, D), jnp.float32),
            jax.ShapeDtypeStruct((NPAD, 1), jnp.float32),
        ],
    )(x_pad, W, deg0, deg1)


@functools.partial(
    pl.kernel,
    mesh=_mesh,
    out_type=jax.ShapeDtypeStruct((2, NPAD, D), jnp.float32),
    scratch_types=[
        pltpu.VMEM((NBLK0, 128), jnp.int32),
        pltpu.VMEM((NBLK0, 128), jnp.int32),
        pltpu.VMEM((128, D), jnp.float32),
        pltpu.SemaphoreType.DMA,
        pltpu.VMEM_SHARED((NPAD, D), jnp.float32),
    ],
    compiler_params=pltpu.CompilerParams(needs_layout_passes=False),
)
def _msg_kernel(src_hbm, dst_hbm, scaled_hbm, accs_out,
                src_v, dst_v, rows, sem, acc_sh):
    c = lax.axis_index("c")
    s = lax.axis_index("s")

    @pl.when(c == 0)
    def _():
        pltpu.sync_copy(src_hbm.at[pl.ds(s * NBLK0, NBLK0)], src_v)
        pltpu.sync_copy(dst_hbm.at[pl.ds(s * NBLK0, NBLK0)], dst_v)

    @pl.when(c == 1)
    def _():
        base = 16 * NBLK0 + s * NBLK1
        pltpu.sync_copy(src_hbm.at[pl.ds(base, NBLK1)],
                        src_v.at[pl.ds(0, NBLK1)])
        pltpu.sync_copy(dst_hbm.at[pl.ds(base, NBLK1)],
                        dst_v.at[pl.ds(0, NBLK1)])

    @pl.loop(0, ROWS_PER_TILE // 128)
    def _(i):
        pltpu.sync_copy(
            scaled_hbm.at[pl.ds(N, 128)],
            acc_sh.at[pl.ds(s * ROWS_PER_TILE + i * 128, 128)])
    plsc.subcore_barrier()

    nblk = jnp.where(c == 0, NBLK0, NBLK1)

    @pl.loop(0, nblk)
    def _(j):
        pltpu.async_copy(scaled_hbm.at[src_v.at[j]], rows, sem).wait()
        pltpu.sync_copy(rows, acc_sh.at[dst_v.at[j]], add=True)

    plsc.subcore_barrier()
    pltpu.sync_copy(
        acc_sh.at[pl.ds(s * ROWS_PER_TILE, ROWS_PER_TILE)],
        accs_out.at[c, pl.ds(s * ROWS_PER_TILE, ROWS_PER_TILE)])


def _combine_body(a0_ref, a1_ref, sc_ref, dis_ref, b_ref, out_ref):
    acc = a0_ref[...] + a1_ref[...] + sc_ref[...]
    out_ref[...] = acc * dis_ref[...] + b_ref[...]


def _combine_call(a0, a1, scaled, dis, b2d):
    blk = 400
    grid = N // blk
    row_spec = pl.BlockSpec((blk, D), lambda i: (i, 0))
    return pl.pallas_call(
        _combine_body,
        grid=(grid,),
        in_specs=[
            row_spec, row_spec, row_spec,
            pl.BlockSpec((blk, 1), lambda i: (i, 0)),
            pl.BlockSpec((1, D), lambda i: (0, 0)),
        ],
        out_specs=row_spec,
        out_shape=jax.ShapeDtypeStruct((N, D), jnp.float32),
    )(a0, a1, scaled, dis, b2d)


def kernel(x, edge_index, W, b):
    src = edge_index[0].astype(jnp.int32)
    dst = edge_index[1].astype(jnp.int32)
    padv = jnp.full((EPAD - E,), PADIDX, jnp.int32)
    src_p = jnp.concatenate([src, padv])
    dst_p = jnp.concatenate([dst, padv])
    packed = jnp.bitwise_or(jnp.left_shift(dst_p, 14), src_p)
    pk_deg = packed.reshape(32, DEG_EPW)

    x_pad = jnp.zeros((NPAD, D), jnp.float32).at[:N].set(x)

    deg2 = _deg_kernel(pk_deg)
    deg0 = deg2[0].reshape(NPAD, 1)
    deg1 = deg2[1].reshape(NPAD, 1)

    scaled, dis = _scale_call(x_pad, W, deg0, deg1)

    accs = _msg_kernel(src_p.reshape(EROWS, 128),
                       dst_p.reshape(EROWS, 128), scaled)

    out = _combine_call(accs[0], accs[1], scaled, dis, b.reshape(1, D))
    return out


# speedup vs baseline: 1.3140x; 1.3140x over previous
import functools
import jax
import jax.numpy as jnp
from jax import lax
from jax.experimental import pallas as pl
from jax.experimental.pallas import tpu as pltpu
from jax.experimental.pallas import tpu_sc as plsc

N = 10000
E = 320000
D = 128
NPAD = 10240
ROWS_PER_TILE = NPAD // 16
NBLK = 79
EPW = NBLK * 128
EPAD = 32 * EPW
DEG_EPW = EPAD // 32
PADIDX = NPAD - 1

_mesh = plsc.VectorSubcoreMesh(core_axis_name="c", subcore_axis_name="s")


@functools.partial(
    pl.kernel,
    mesh=_mesh,
    out_type=jax.ShapeDtypeStruct((2, NPAD), jnp.float32),
    scratch_types=[
        pltpu.VMEM((DEG_EPW,), jnp.int32),
        pltpu.VMEM((NPAD,), jnp.float32),
        pltpu.VMEM((ROWS_PER_TILE,), jnp.float32),
        pltpu.VMEM((ROWS_PER_TILE,), jnp.float32),
        pltpu.VMEM_SHARED((16, NPAD), jnp.float32),
    ],
    compiler_params=pltpu.CompilerParams(needs_layout_passes=False),
)
def _deg_kernel(pk_hbm, deg_out, pk_v, deg_v, tmp_v, red_v, slots):
    c = lax.axis_index("c")
    s = lax.axis_index("s")
    wid = c * 16 + s
    pltpu.sync_copy(pk_hbm.at[wid], pk_v)

    zeros16 = jnp.zeros((16,), jnp.float32)

    @pl.loop(0, NPAD // 16)
    def _(i):
        deg_v[pl.ds(i * 16, 16)] = zeros16

    @pl.loop(0, DEG_EPW // 16)
    def _(i):
        idx = lax.shift_right_logical(pk_v[pl.ds(i * 16, 16)], 14)
        cnt, last = plsc.scan_count(idx)
        plsc.addupdate_scatter(
            deg_v, [idx], cnt.astype(jnp.float32), mask=last)

    pltpu.sync_copy(deg_v, slots.at[s])
    plsc.subcore_barrier()

    @pl.loop(0, ROWS_PER_TILE // 16)
    def _(i):
        red_v[pl.ds(i * 16, 16)] = zeros16

    @pl.loop(0, 16)
    def _(t):
        pltpu.sync_copy(slots.at[t, pl.ds(s * ROWS_PER_TILE, ROWS_PER_TILE)],
                        tmp_v)

        @pl.loop(0, ROWS_PER_TILE // 16)
        def _(i):
            sl = pl.ds(i * 16, 16)
            red_v[sl] = red_v[sl] + tmp_v[sl]

    pltpu.sync_copy(red_v,
                    deg_out.at[c, pl.ds(s * ROWS_PER_TILE, ROWS_PER_TILE)])


def _scale_body(x_ref, w_ref, d0_ref, d1_ref, scaled_ref, dis_ref):
    deg = d0_ref[...] + d1_ref[...] + 1.0
    dis = lax.rsqrt(deg)
    xw = jnp.dot(x_ref[...], w_ref[...], preferred_element_type=jnp.float32)
    scaled_ref[...] = xw * dis
    dis_ref[...] = dis


def _scale_call(x_pad, W, deg0, deg1):
    blk = 512
    grid = NPAD // blk
    return pl.pallas_call(
        _scale_body,
        grid=(grid,),
        in_specs=[
            pl.BlockSpec((blk, D), lambda i: (i, 0)),
            pl.BlockSpec((D, D), lambda i: (0, 0)),
            pl.BlockSpec((blk, 1), lambda i: (i, 0)),
            pl.BlockSpec((blk, 1), lambda i: (i, 0)),
        ],
        out_specs=[
            pl.BlockSpec((blk, D), lambda i: (i, 0)),
            pl.BlockSpec((blk, 1), lambda i: (i, 0)),
        ],
        out_shape=[
            jax.ShapeDtypeStruct((NPAD, D), jnp.float32),
            jax.ShapeDtypeStruct((NPAD, 1), jnp.float32),
        ],
    )(x_pad, W, deg0, deg1)


@functools.partial(
    pl.kernel,
    mesh=_mesh,
    out_type=jax.ShapeDtypeStruct((2, NPAD, D), jnp.float32),
    scratch_types=[
        pltpu.VMEM((NBLK, 128), jnp.int32),
        pltpu.VMEM((NBLK, 128), jnp.int32),
        pltpu.VMEM((128, D), jnp.float32),
        pltpu.SemaphoreType.DMA,
        pltpu.VMEM_SHARED((NPAD, D), jnp.float32),
    ],
    compiler_params=pltpu.CompilerParams(needs_layout_passes=False),
)
def _msg_kernel(src_hbm, dst_hbm, scaled_hbm, accs_out,
                src_v, dst_v, rows, sem, acc_sh):
    c = lax.axis_index("c")
    s = lax.axis_index("s")
    wid = c * 16 + s
    pltpu.sync_copy(src_hbm.at[wid], src_v)
    pltpu.sync_copy(dst_hbm.at[wid], dst_v)

    @pl.loop(0, ROWS_PER_TILE // 128)
    def _(i):
        pltpu.sync_copy(
            scaled_hbm.at[pl.ds(N, 128)],
            acc_sh.at[pl.ds(s * ROWS_PER_TILE + i * 128, 128)])
    plsc.subcore_barrier()

    @pl.loop(0, NBLK)
    def _(j):
        pltpu.async_copy(scaled_hbm.at[src_v.at[j]], rows, sem).wait()
        pltpu.sync_copy(rows, acc_sh.at[dst_v.at[j]], add=True)

    plsc.subcore_barrier()
    pltpu.sync_copy(
        acc_sh.at[pl.ds(s * ROWS_PER_TILE, ROWS_PER_TILE)],
        accs_out.at[c, pl.ds(s * ROWS_PER_TILE, ROWS_PER_TILE)])


def _combine_body(a0_ref, a1_ref, sc_ref, dis_ref, b_ref, out_ref):
    acc = a0_ref[...] + a1_ref[...] + sc_ref[...]
    out_ref[...] = acc * dis_ref[...] + b_ref[...]


def _combine_call(a0, a1, scaled, dis, b2d):
    blk = 400
    grid = N // blk
    row_spec = pl.BlockSpec((blk, D), lambda i: (i, 0))
    return pl.pallas_call(
        _combine_body,
        grid=(grid,),
        in_specs=[
            row_spec, row_spec, row_spec,
            pl.BlockSpec((blk, 1), lambda i: (i, 0)),
            pl.BlockSpec((1, D), lambda i: (0, 0)),
        ],
        out_specs=row_spec,
        out_shape=jax.ShapeDtypeStruct((N, D), jnp.float32),
    )(a0, a1, scaled, dis, b2d)


def kernel(x, edge_index, W, b):
    src = edge_index[0].astype(jnp.int32)
    dst = edge_index[1].astype(jnp.int32)
    padv = jnp.full((EPAD - E,), PADIDX, jnp.int32)
    src_p = jnp.concatenate([src, padv])
    dst_p = jnp.concatenate([dst, padv])
    packed = jnp.bitwise_or(jnp.left_shift(dst_p, 14), src_p)
    pk_deg = packed.reshape(32, DEG_EPW)

    x_pad = jnp.zeros((NPAD, D), jnp.float32).at[:N].set(x)

    deg2 = _deg_kernel(pk_deg)
    deg0 = deg2[0].reshape(NPAD, 1)
    deg1 = deg2[1].reshape(NPAD, 1)

    scaled, dis = _scale_call(x_pad, W, deg0, deg1)

    accs = _msg_kernel(src_p.reshape(32, NBLK, 128),
                       dst_p.reshape(32, NBLK, 128), scaled)

    out = _combine_call(accs[0], accs[1], scaled, dis, b.reshape(1, D))
    return out
